# baseline (speedup 1.0000x reference)
import functools

import jax
import jax.numpy as jnp
from jax import lax
from jax.experimental import pallas as pl
from jax.experimental.pallas import tpu as pltpu
from jax.experimental.pallas import tpu_sc as plsc

N = 10000
E = 160000
D = 256
HALF = 128
NC, NS = 2, 16
NW = NC * NS
CH = 128
EP = 163840
NROW = EP // CH
RPW = NROW // NW
CH2 = 64
NROW2 = EP // CH2
RPC = NROW2 // NS
NB = 4
RPS = 640
ACC = NS * RPS
DUMMY = N
R = 1000


def _sc_mesh():
    return plsc.VectorSubcoreMesh(core_axis_name="c", subcore_axis_name="s",
                                  num_cores=NC, num_subcores=NS)


def _zero_1d(ref, n):
    def body(i, carry):
        ref[pl.ds(i * 16, 16)] = jnp.zeros((16,), jnp.float32)
        return carry
    lax.fori_loop(0, n // 16, body, 0)


def _zero_2d(ref, rows, cols):
    z = jnp.zeros((16,), jnp.float32)
    for r in range(rows):
        for kk in range(cols // 16):
            ref[r, pl.ds(kk * 16, 16)] = z


@functools.cache
def _build_deg_kernel():
    return functools.partial(
        pl.kernel,
        out_type=jax.ShapeDtypeStruct((NC, ACC), jnp.float32),
        mesh=_sc_mesh(),
        scratch_types=[
            pltpu.VMEM((RPW, CH), jnp.int32),
            pltpu.VMEM((CH,), jnp.float32),
            pltpu.VMEM((RPS,), jnp.float32),
            pltpu.VMEM_SHARED((ACC,), jnp.float32),
            pltpu.SemaphoreType.DMA,
        ],
    )(_deg_body)


def _deg_body(dst_hbm, out_hbm, idx_v, ones_v, z_v, acc_sh, dsem):
    c = lax.axis_index("c")
    s = lax.axis_index("s")
    wid = c * NS + s

    def ob(i, carry):
        ones_v[pl.ds(i * 16, 16)] = jnp.ones((16,), jnp.float32)
        return carry
    lax.fori_loop(0, CH // 16, ob, 0)
    _zero_1d(z_v, RPS)
    pltpu.sync_copy(z_v, acc_sh.at[pl.ds(s * RPS, RPS)])
    pltpu.sync_copy(dst_hbm.at[pl.ds(wid * RPW, RPW)], idx_v)
    plsc.subcore_barrier()

    def eb(j, carry):
        pltpu.async_copy(ones_v, acc_sh.at[idx_v.at[j]], dsem, add=True)
        return carry
    lax.fori_loop(0, RPW, eb, 0)

    def db(j, carry):
        pltpu.make_async_copy(ones_v, acc_sh.at[idx_v.at[0]], dsem).wait()
        return carry
    lax.fori_loop(0, RPW, db, 0)
    plsc.subcore_barrier()
    pltpu.sync_copy(acc_sh.at[pl.ds(s * RPS, RPS)],
                    out_hbm.at[c].at[pl.ds(s * RPS, RPS)])


@functools.cache
def _build_agg_kernel():
    return functools.partial(
        pl.kernel,
        out_type=jax.ShapeDtypeStruct((NC, N, HALF), jnp.float32),
        mesh=_sc_mesh(),
        scratch_types=[
            [pltpu.VMEM((RPC // 10, CH2), jnp.int32) for _ in range(2)],
            [pltpu.VMEM((RPC // 10, CH2), jnp.int32) for _ in range(2)],
            [pltpu.VMEM((CH2, HALF), jnp.float32) for _ in range(NB)],
            pltpu.VMEM_SHARED((ACC, HALF), jnp.float32),
            [pltpu.SemaphoreType.DMA for _ in range(NB)],
            [pltpu.SemaphoreType.DMA for _ in range(NB)],
            [pltpu.SemaphoreType.DMA for _ in range(2)],
        ],
    )(_agg_body)


def _agg_body(tab_hbm, src_hbm, dst_hbm, out_hbm, srcb, dstb, bufs,
              acc_sh, gsems, ssems, isems):
    c = lax.axis_index("c")
    s = lax.axis_index("s")

    NSTG = 10
    SRW = RPC // NSTG

    def idx_fetch(h):
        b = h % 2
        off = s * RPC + h * SRW
        pltpu.async_copy(src_hbm.at[pl.ds(off, SRW)], srcb[b], isems[b])
        pltpu.async_copy(dst_hbm.at[pl.ds(off, SRW)], dstb[b], isems[b])

    def idx_wait(h):
        b = h % 2
        off = s * RPC + h * SRW
        pltpu.make_async_copy(src_hbm.at[pl.ds(off, SRW)], srcb[b],
                              isems[b]).wait()
        pltpu.make_async_copy(dst_hbm.at[pl.ds(off, SRW)], dstb[b],
                              isems[b]).wait()

    def g_start(b, r, k):
        pltpu.async_copy(tab_hbm.at[c].at[srcb[b].at[r]], bufs[k], gsems[k])

    def g_wait(k):
        pltpu.make_async_copy(tab_hbm.at[c].at[srcb[0].at[0]], bufs[k],
                              gsems[k]).wait()

    def s_start(b, r, k):
        pltpu.async_copy(bufs[k], acc_sh.at[dstb[b].at[r]], ssems[k],
                         add=True)

    def s_wait(k):
        pltpu.make_async_copy(bufs[k], acc_sh.at[dstb[0].at[0]],
                              ssems[k]).wait()

    idx_fetch(0)
    _zero_2d(bufs[NB - 1], CH2, HALF)
    for i in range(RPS // CH2):
        pltpu.async_copy(bufs[NB - 1],
                         acc_sh.at[pl.ds(s * RPS + i * CH2, CH2)],
                         ssems[NB - 1])
    idx_wait(0)
    for k in range(NB - 1):
        g_start(0, k, k)
    for i in range(RPS // CH2):
        pltpu.make_async_copy(bufs[NB - 1],
                              acc_sh.at[pl.ds(s * RPS, CH2)],
                              ssems[NB - 1]).wait()
    plsc.subcore_barrier()
    g_start(0, NB - 1, NB - 1)
    for h in range(NSTG):
        b = h % 2
        if h + 1 < NSTG:
            idx_fetch(h + 1)

        def rb(i, carry):
            base = i * NB
            for k in range(NB):
                g_wait(k)
                s_start(b, base + k, k)
            for k in range(NB):
                nxt = base + NB + k
                @pl.when(nxt < SRW)
                def _():
                    s_wait(k)
                    g_start(b, nxt, k)
            return carry
        lax.fori_loop(0, SRW // NB, rb, 0)
        if h + 1 < NSTG:
            idx_wait(h + 1)
            for k in range(NB):
                s_wait(k)
                g_start((h + 1) % 2, k, k)
        else:
            for k in range(NB):
                s_wait(k)
    plsc.subcore_barrier()

    @pl.when(s < NS - 1)
    def _copy_full():
        pltpu.sync_copy(acc_sh.at[pl.ds(s * RPS, RPS)],
                        out_hbm.at[c].at[pl.ds(s * RPS, RPS)])

    @pl.when(s == NS - 1)
    def _copy_tail():
        pltpu.sync_copy(acc_sh.at[pl.ds((NS - 1) * RPS, N - (NS - 1) * RPS)],
                        out_hbm.at[c].at[pl.ds((NS - 1) * RPS,
                                               N - (NS - 1) * RPS)])


@functools.cache
def _build_agg1_kernel():
    return functools.partial(
        pl.kernel,
        out_type=jax.ShapeDtypeStruct((NC, ACC), jnp.float32),
        mesh=_sc_mesh(),
        scratch_types=[
            pltpu.VMEM((RPW, CH), jnp.int32),
            pltpu.VMEM((RPW, CH), jnp.int32),
            [pltpu.VMEM((CH,), jnp.float32) for _ in range(NB)],
            pltpu.VMEM((RPS,), jnp.float32),
            pltpu.VMEM_SHARED((ACC,), jnp.float32),
            [pltpu.SemaphoreType.DMA for _ in range(NB)],
            [pltpu.SemaphoreType.DMA for _ in range(NB)],
        ],
    )(_agg1_body)


def _agg1_body(g2_hbm, src_hbm, dst_hbm, out_hbm, src_v, dst_v,
               vals, z_v, acc_sh, gsems, ssems):
    c = lax.axis_index("c")
    s = lax.axis_index("s")
    wid = c * NS + s

    _zero_1d(z_v, RPS)
    pltpu.async_copy(z_v, acc_sh.at[pl.ds(s * RPS, RPS)], ssems[0])
    pltpu.sync_copy(src_hbm.at[pl.ds(wid * RPW, RPW)], src_v)
    pltpu.sync_copy(dst_hbm.at[pl.ds(wid * RPW, RPW)], dst_v)
    pltpu.make_async_copy(z_v, acc_sh.at[pl.ds(s * RPS, RPS)],
                          ssems[0]).wait()
    plsc.subcore_barrier()

    def g_start(r, k):
        pltpu.async_copy(g2_hbm.at[src_v.at[r]], vals[k], gsems[k])

    def g_wait(k):
        pltpu.make_async_copy(g2_hbm.at[src_v.at[0]], vals[k],
                              gsems[k]).wait()

    def s_start(r, k):
        pltpu.async_copy(vals[k], acc_sh.at[dst_v.at[r]], ssems[k],
                         add=True)

    def s_wait(k):
        pltpu.make_async_copy(vals[k], acc_sh.at[dst_v.at[0]],
                              ssems[k]).wait()

    for k in range(NB):
        g_start(k, k)

    def eb(i, carry):
        base = i * NB
        for k in range(NB):
            g_wait(k)
            s_start(base + k, k)
        for k in range(NB):
            nxt = base + NB + k
            @pl.when(nxt < RPW)
            def _():
                s_wait(k)
                g_start(nxt, k)
        return carry
    lax.fori_loop(0, RPW // NB, eb, 0)
    for k in range(NB):
        s_wait(k)
    plsc.subcore_barrier()
    pltpu.sync_copy(acc_sh.at[pl.ds(s * RPS, RPS)],
                    out_hbm.at[c].at[pl.ds(s * RPS, RPS)])


def _mm_body(x_ref, w_ref, d0_ref, d1_ref, tab_ref, dinv_ref):
    deg = d0_ref[...] + d1_ref[...] + 1.0
    dinv = lax.rsqrt(deg)
    h = jnp.dot(x_ref[...], w_ref[...], preferred_element_type=jnp.float32)
    g = h * dinv
    tab_ref[0] = g[:, :HALF]
    tab_ref[1] = g[:, HALF:]
    dinv_ref[...] = dinv


def _ep_body(s_ref, g_ref, dinv_ref, b1_ref, w2_ref, g2_ref):
    sfull = jnp.concatenate([s_ref[0], s_ref[1]], axis=1)
    gfull = jnp.concatenate([g_ref[0], g_ref[1]], axis=1)
    dinv = dinv_ref[...]
    o1 = jnp.maximum(dinv * (sfull + gfull) + b1_ref[...], 0.0)
    h2 = jnp.dot(o1, w2_ref[...], preferred_element_type=jnp.float32)
    g2_ref[...] = h2 * dinv


def _fin_body(g2_ref, sa_ref, sb_ref, dinv_ref, b2_ref, out_ref):
    t = dinv_ref[...] * (sa_ref[...] + sb_ref[...] + g2_ref[...]) + b2_ref[...]
    out_ref[...] = jax.nn.sigmoid(t)


def kernel(x, edge_index, W1, b1, W2, b2):
    src = edge_index[0].astype(jnp.int32)
    dst = edge_index[1].astype(jnp.int32)
    pad = EP - E
    srcp = jnp.concatenate([src, jnp.zeros((pad,), jnp.int32)]).reshape(NROW, CH)
    dstp = jnp.concatenate([dst, jnp.full((pad,), DUMMY, jnp.int32)]).reshape(NROW, CH)

    deg2 = _build_deg_kernel()(dstp)
    d0 = deg2[0].reshape(ACC, 1)
    d1 = deg2[1].reshape(ACC, 1)

    tab, dinv = pl.pallas_call(
        _mm_body,
        grid=(N // R,),
        in_specs=[
            pl.BlockSpec((R, D), lambda i: (i, 0)),
            pl.BlockSpec((D, D), lambda i: (0, 0)),
            pl.BlockSpec((R, 1), lambda i: (i, 0)),
            pl.BlockSpec((R, 1), lambda i: (i, 0)),
        ],
        out_specs=[
            pl.BlockSpec((2, R, HALF), lambda i: (0, i, 0)),
            pl.BlockSpec((R, 1), lambda i: (i, 0)),
        ],
        out_shape=[
            jax.ShapeDtypeStruct((2, N, HALF), jnp.float32),
            jax.ShapeDtypeStruct((N, 1), jnp.float32),
        ],
    )(x, W1, d0, d1)

    sacc = _build_agg_kernel()(tab, srcp.reshape(NROW2, CH2),
                               dstp.reshape(NROW2, CH2))

    g2 = pl.pallas_call(
        _ep_body,
        grid=(N // R,),
        in_specs=[
            pl.BlockSpec((2, R, HALF), lambda i: (0, i, 0)),
            pl.BlockSpec((2, R, HALF), lambda i: (0, i, 0)),
            pl.BlockSpec((R, 1), lambda i: (i, 0)),
            pl.BlockSpec((1, D), lambda i: (0, 0)),
            pl.BlockSpec((D, 1), lambda i: (0, 0)),
        ],
        out_specs=pl.BlockSpec((R, 1), lambda i: (i, 0)),
        out_shape=jax.ShapeDtypeStruct((N, 1), jnp.float32),
    )(sacc, tab, dinv, b1.reshape(1, D), W2)

    s2 = _build_agg1_kernel()(g2.reshape(N), srcp, dstp)

    out = pl.pallas_call(
        _fin_body,
        grid=(N // R,),
        in_specs=[
            pl.BlockSpec((R, 1), lambda i: (i, 0)),
            pl.BlockSpec((R, 1), lambda i: (i, 0)),
            pl.BlockSpec((R, 1), lambda i: (i, 0)),
            pl.BlockSpec((R, 1), lambda i: (i, 0)),
            pl.BlockSpec((1, 1), lambda i: (0, 0)),
        ],
        out_specs=pl.BlockSpec((R, 1), lambda i: (i, 0)),
        out_shape=jax.ShapeDtypeStruct((N, 1), jnp.float32),
    )(g2, s2[0].reshape(ACC, 1), s2[1].reshape(ACC, 1), dinv,
      b2.reshape(1, 1))
    return out


# speedup vs baseline: 1.0170x; 1.0170x over previous
import functools

import jax
import jax.numpy as jnp
from jax import lax
from jax.experimental import pallas as pl
from jax.experimental.pallas import tpu as pltpu
from jax.experimental.pallas import tpu_sc as plsc

N = 10000
E = 160000
D = 256
HALF = 128
NC, NS = 2, 16
NW = NC * NS
CH = 128
EP = 163840
NROW = EP // CH
RPW = NROW // NW
CH2 = 64
NROW2 = EP // CH2
RPC = NROW2 // NS
NB = 4
RPS = 640
ACC = NS * RPS
DUMMY = N
R = 2000


def _sc_mesh():
    return plsc.VectorSubcoreMesh(core_axis_name="c", subcore_axis_name="s",
                                  num_cores=NC, num_subcores=NS)


def _zero_1d(ref, n):
    def body(i, carry):
        ref[pl.ds(i * 16, 16)] = jnp.zeros((16,), jnp.float32)
        return carry
    lax.fori_loop(0, n // 16, body, 0)


def _zero_2d(ref, rows, cols):
    z = jnp.zeros((16,), jnp.float32)
    for r in range(rows):
        for kk in range(cols // 16):
            ref[r, pl.ds(kk * 16, 16)] = z


@functools.cache
def _build_deg_kernel():
    return functools.partial(
        pl.kernel,
        out_type=jax.ShapeDtypeStruct((NC, ACC), jnp.float32),
        mesh=_sc_mesh(),
        scratch_types=[
            pltpu.VMEM((RPW, CH), jnp.int32),
            pltpu.VMEM((CH,), jnp.float32),
            pltpu.VMEM((RPS,), jnp.float32),
            pltpu.VMEM_SHARED((ACC,), jnp.float32),
            pltpu.SemaphoreType.DMA,
        ],
    )(_deg_body)


def _deg_body(dst_hbm, out_hbm, idx_v, ones_v, z_v, acc_sh, dsem):
    c = lax.axis_index("c")
    s = lax.axis_index("s")
    wid = c * NS + s

    def ob(i, carry):
        ones_v[pl.ds(i * 16, 16)] = jnp.ones((16,), jnp.float32)
        return carry
    lax.fori_loop(0, CH // 16, ob, 0)
    _zero_1d(z_v, RPS)
    pltpu.sync_copy(z_v, acc_sh.at[pl.ds(s * RPS, RPS)])
    pltpu.sync_copy(dst_hbm.at[pl.ds(wid * RPW, RPW)], idx_v)
    plsc.subcore_barrier()

    def eb(j, carry):
        pltpu.async_copy(ones_v, acc_sh.at[idx_v.at[j]], dsem, add=True)
        return carry
    lax.fori_loop(0, RPW, eb, 0)

    def db(j, carry):
        pltpu.make_async_copy(ones_v, acc_sh.at[idx_v.at[0]], dsem).wait()
        return carry
    lax.fori_loop(0, RPW, db, 0)
    plsc.subcore_barrier()
    pltpu.sync_copy(acc_sh.at[pl.ds(s * RPS, RPS)],
                    out_hbm.at[c].at[pl.ds(s * RPS, RPS)])


@functools.cache
def _build_agg_kernel():
    return functools.partial(
        pl.kernel,
        out_type=jax.ShapeDtypeStruct((NC, N, HALF), jnp.float32),
        mesh=_sc_mesh(),
        scratch_types=[
            [pltpu.VMEM((RPC // 10, CH2), jnp.int32) for _ in range(2)],
            [pltpu.VMEM((RPC // 10, CH2), jnp.int32) for _ in range(2)],
            [pltpu.VMEM((CH2, HALF), jnp.float32) for _ in range(NB)],
            pltpu.VMEM_SHARED((ACC, HALF), jnp.float32),
            [pltpu.SemaphoreType.DMA for _ in range(NB)],
            [pltpu.SemaphoreType.DMA for _ in range(NB)],
            [pltpu.SemaphoreType.DMA for _ in range(2)],
        ],
    )(_agg_body)


def _agg_body(tab_hbm, src_hbm, dst_hbm, out_hbm, srcb, dstb, bufs,
              acc_sh, gsems, ssems, isems):
    c = lax.axis_index("c")
    s = lax.axis_index("s")

    NSTG = 10
    SRW = RPC // NSTG

    def idx_fetch(h):
        b = h % 2
        off = s * RPC + h * SRW
        pltpu.async_copy(src_hbm.at[pl.ds(off, SRW)], srcb[b], isems[b])
        pltpu.async_copy(dst_hbm.at[pl.ds(off, SRW)], dstb[b], isems[b])

    def idx_wait(h):
        b = h % 2
        off = s * RPC + h * SRW
        pltpu.make_async_copy(src_hbm.at[pl.ds(off, SRW)], srcb[b],
                              isems[b]).wait()
        pltpu.make_async_copy(dst_hbm.at[pl.ds(off, SRW)], dstb[b],
                              isems[b]).wait()

    def g_start(b, r, k):
        pltpu.async_copy(tab_hbm.at[c].at[srcb[b].at[r]], bufs[k], gsems[k])

    def g_wait(k):
        pltpu.make_async_copy(tab_hbm.at[c].at[srcb[0].at[0]], bufs[k],
                              gsems[k]).wait()

    def s_start(b, r, k):
        pltpu.async_copy(bufs[k], acc_sh.at[dstb[b].at[r]], ssems[k],
                         add=True)

    def s_wait(k):
        pltpu.make_async_copy(bufs[k], acc_sh.at[dstb[0].at[0]],
                              ssems[k]).wait()

    idx_fetch(0)
    _zero_2d(bufs[NB - 1], CH2, HALF)
    for i in range(RPS // CH2):
        pltpu.async_copy(bufs[NB - 1],
                         acc_sh.at[pl.ds(s * RPS + i * CH2, CH2)],
                         ssems[NB - 1])
    idx_wait(0)
    for k in range(NB - 1):
        g_start(0, k, k)
    for i in range(RPS // CH2):
        pltpu.make_async_copy(bufs[NB - 1],
                              acc_sh.at[pl.ds(s * RPS, CH2)],
                              ssems[NB - 1]).wait()
    plsc.subcore_barrier()
    g_start(0, NB - 1, NB - 1)
    for h in range(NSTG):
        b = h % 2
        if h + 1 < NSTG:
            idx_fetch(h + 1)

        def rb(i, carry):
            base = i * NB
            for k in range(NB):
                g_wait(k)
                s_start(b, base + k, k)
            for k in range(NB):
                nxt = base + NB + k
                @pl.when(nxt < SRW)
                def _():
                    s_wait(k)
                    g_start(b, nxt, k)
            return carry
        lax.fori_loop(0, SRW // NB, rb, 0)
        if h + 1 < NSTG:
            idx_wait(h + 1)
            for k in range(NB):
                s_wait(k)
                g_start((h + 1) % 2, k, k)
        else:
            for k in range(NB):
                s_wait(k)
    plsc.subcore_barrier()

    @pl.when(s < NS - 1)
    def _copy_full():
        pltpu.sync_copy(acc_sh.at[pl.ds(s * RPS, RPS)],
                        out_hbm.at[c].at[pl.ds(s * RPS, RPS)])

    @pl.when(s == NS - 1)
    def _copy_tail():
        pltpu.sync_copy(acc_sh.at[pl.ds((NS - 1) * RPS, N - (NS - 1) * RPS)],
                        out_hbm.at[c].at[pl.ds((NS - 1) * RPS,
                                               N - (NS - 1) * RPS)])


@functools.cache
def _build_agg1_kernel():
    return functools.partial(
        pl.kernel,
        out_type=jax.ShapeDtypeStruct((NC, ACC), jnp.float32),
        mesh=_sc_mesh(),
        scratch_types=[
            pltpu.VMEM((RPW, CH), jnp.int32),
            pltpu.VMEM((RPW, CH), jnp.int32),
            [pltpu.VMEM((CH,), jnp.float32) for _ in range(NB)],
            pltpu.VMEM((RPS,), jnp.float32),
            pltpu.VMEM_SHARED((ACC,), jnp.float32),
            [pltpu.SemaphoreType.DMA for _ in range(NB)],
            [pltpu.SemaphoreType.DMA for _ in range(NB)],
        ],
    )(_agg1_body)


def _agg1_body(g2_hbm, src_hbm, dst_hbm, out_hbm, src_v, dst_v,
               vals, z_v, acc_sh, gsems, ssems):
    c = lax.axis_index("c")
    s = lax.axis_index("s")
    wid = c * NS + s

    _zero_1d(z_v, RPS)
    pltpu.async_copy(z_v, acc_sh.at[pl.ds(s * RPS, RPS)], ssems[0])
    pltpu.sync_copy(src_hbm.at[pl.ds(wid * RPW, RPW)], src_v)
    pltpu.sync_copy(dst_hbm.at[pl.ds(wid * RPW, RPW)], dst_v)
    pltpu.make_async_copy(z_v, acc_sh.at[pl.ds(s * RPS, RPS)],
                          ssems[0]).wait()
    plsc.subcore_barrier()

    def g_start(r, k):
        pltpu.async_copy(g2_hbm.at[src_v.at[r]], vals[k], gsems[k])

    def g_wait(k):
        pltpu.make_async_copy(g2_hbm.at[src_v.at[0]], vals[k],
                              gsems[k]).wait()

    def s_start(r, k):
        pltpu.async_copy(vals[k], acc_sh.at[dst_v.at[r]], ssems[k],
                         add=True)

    def s_wait(k):
        pltpu.make_async_copy(vals[k], acc_sh.at[dst_v.at[0]],
                              ssems[k]).wait()

    for k in range(NB):
        g_start(k, k)

    def eb(i, carry):
        base = i * NB
        for k in range(NB):
            g_wait(k)
            s_start(base + k, k)
        for k in range(NB):
            nxt = base + NB + k
            @pl.when(nxt < RPW)
            def _():
                s_wait(k)
                g_start(nxt, k)
        return carry
    lax.fori_loop(0, RPW // NB, eb, 0)
    for k in range(NB):
        s_wait(k)
    plsc.subcore_barrier()
    pltpu.sync_copy(acc_sh.at[pl.ds(s * RPS, RPS)],
                    out_hbm.at[c].at[pl.ds(s * RPS, RPS)])


def _mm_body(x_ref, w_ref, d0_ref, d1_ref, tab_ref, dinv_ref):
    deg = d0_ref[...] + d1_ref[...] + 1.0
    dinv = lax.rsqrt(deg)
    h = jnp.dot(x_ref[...], w_ref[...], preferred_element_type=jnp.float32)
    g = h * dinv
    tab_ref[0] = g[:, :HALF]
    tab_ref[1] = g[:, HALF:]
    dinv_ref[...] = dinv


def _ep_body(s_ref, g_ref, dinv_ref, b1_ref, w2_ref, g2_ref):
    sfull = jnp.concatenate([s_ref[0], s_ref[1]], axis=1)
    gfull = jnp.concatenate([g_ref[0], g_ref[1]], axis=1)
    dinv = dinv_ref[...]
    o1 = jnp.maximum(dinv * (sfull + gfull) + b1_ref[...], 0.0)
    h2 = jnp.dot(o1, w2_ref[...], preferred_element_type=jnp.float32)
    g2_ref[...] = h2 * dinv


def _fin_body(g2_ref, sa_ref, sb_ref, dinv_ref, b2_ref, out_ref):
    t = dinv_ref[...] * (sa_ref[...] + sb_ref[...] + g2_ref[...]) + b2_ref[...]
    out_ref[...] = jax.nn.sigmoid(t)


def kernel(x, edge_index, W1, b1, W2, b2):
    src = edge_index[0].astype(jnp.int32)
    dst = edge_index[1].astype(jnp.int32)
    pad = EP - E
    srcp = jnp.concatenate([src, jnp.zeros((pad,), jnp.int32)]).reshape(NROW, CH)
    dstp = jnp.concatenate([dst, jnp.full((pad,), DUMMY, jnp.int32)]).reshape(NROW, CH)

    deg2 = _build_deg_kernel()(dstp)
    d0 = deg2[0].reshape(ACC, 1)
    d1 = deg2[1].reshape(ACC, 1)

    tab, dinv = pl.pallas_call(
        _mm_body,
        grid=(N // R,),
        in_specs=[
            pl.BlockSpec((R, D), lambda i: (i, 0)),
            pl.BlockSpec((D, D), lambda i: (0, 0)),
            pl.BlockSpec((R, 1), lambda i: (i, 0)),
            pl.BlockSpec((R, 1), lambda i: (i, 0)),
        ],
        out_specs=[
            pl.BlockSpec((2, R, HALF), lambda i: (0, i, 0)),
            pl.BlockSpec((R, 1), lambda i: (i, 0)),
        ],
        out_shape=[
            jax.ShapeDtypeStruct((2, N, HALF), jnp.float32),
            jax.ShapeDtypeStruct((N, 1), jnp.float32),
        ],
    )(x, W1, d0, d1)

    sacc = _build_agg_kernel()(tab, srcp.reshape(NROW2, CH2),
                               dstp.reshape(NROW2, CH2))

    g2 = pl.pallas_call(
        _ep_body,
        grid=(N // R,),
        in_specs=[
            pl.BlockSpec((2, R, HALF), lambda i: (0, i, 0)),
            pl.BlockSpec((2, R, HALF), lambda i: (0, i, 0)),
            pl.BlockSpec((R, 1), lambda i: (i, 0)),
            pl.BlockSpec((1, D), lambda i: (0, 0)),
            pl.BlockSpec((D, 1), lambda i: (0, 0)),
        ],
        out_specs=pl.BlockSpec((R, 1), lambda i: (i, 0)),
        out_shape=jax.ShapeDtypeStruct((N, 1), jnp.float32),
    )(sacc, tab, dinv, b1.reshape(1, D), W2)

    s2 = _build_agg1_kernel()(g2.reshape(N), srcp, dstp)

    out = pl.pallas_call(
        _fin_body,
        grid=(N // R,),
        in_specs=[
            pl.BlockSpec((R, 1), lambda i: (i, 0)),
            pl.BlockSpec((R, 1), lambda i: (i, 0)),
            pl.BlockSpec((R, 1), lambda i: (i, 0)),
            pl.BlockSpec((R, 1), lambda i: (i, 0)),
            pl.BlockSpec((1, 1), lambda i: (0, 0)),
        ],
        out_specs=pl.BlockSpec((R, 1), lambda i: (i, 0)),
        out_shape=jax.ShapeDtypeStruct((N, 1), jnp.float32),
    )(g2, s2[0].reshape(ACC, 1), s2[1].reshape(ACC, 1), dinv,
      b2.reshape(1, 1))
    return out


# speedup vs baseline: 1.0200x; 1.0029x over previous
import functools

import jax
import jax.numpy as jnp
from jax import lax
from jax.experimental import pallas as pl
from jax.experimental.pallas import tpu as pltpu
from jax.experimental.pallas import tpu_sc as plsc

N = 10000
E = 160000
D = 256
HALF = 128
NC, NS = 2, 16
NW = NC * NS
CH = 128
EP = 163840
NROW = EP // CH
RPW = NROW // NW
CH2 = 64
NROW2 = EP // CH2
RPC = NROW2 // NS
NB = 4
RPS = 640
ACC = NS * RPS
DUMMY = N
R = 5000


def _sc_mesh():
    return plsc.VectorSubcoreMesh(core_axis_name="c", subcore_axis_name="s",
                                  num_cores=NC, num_subcores=NS)


def _zero_1d(ref, n):
    def body(i, carry):
        ref[pl.ds(i * 16, 16)] = jnp.zeros((16,), jnp.float32)
        return carry
    lax.fori_loop(0, n // 16, body, 0)


def _zero_2d(ref, rows, cols):
    z = jnp.zeros((16,), jnp.float32)
    for r in range(rows):
        for kk in range(cols // 16):
            ref[r, pl.ds(kk * 16, 16)] = z


@functools.cache
def _build_deg_kernel():
    return functools.partial(
        pl.kernel,
        out_type=jax.ShapeDtypeStruct((NC, ACC), jnp.float32),
        mesh=_sc_mesh(),
        scratch_types=[
            pltpu.VMEM((RPW, CH), jnp.int32),
            pltpu.VMEM((CH,), jnp.float32),
            pltpu.VMEM((RPS,), jnp.float32),
            pltpu.VMEM_SHARED((ACC,), jnp.float32),
            pltpu.SemaphoreType.DMA,
        ],
    )(_deg_body)


def _deg_body(dst_hbm, out_hbm, idx_v, ones_v, z_v, acc_sh, dsem):
    c = lax.axis_index("c")
    s = lax.axis_index("s")
    wid = c * NS + s

    def ob(i, carry):
        ones_v[pl.ds(i * 16, 16)] = jnp.ones((16,), jnp.float32)
        return carry
    lax.fori_loop(0, CH // 16, ob, 0)
    _zero_1d(z_v, RPS)
    pltpu.sync_copy(z_v, acc_sh.at[pl.ds(s * RPS, RPS)])
    pltpu.sync_copy(dst_hbm.at[pl.ds(wid * RPW, RPW)], idx_v)
    plsc.subcore_barrier()

    def eb(j, carry):
        pltpu.async_copy(ones_v, acc_sh.at[idx_v.at[j]], dsem, add=True)
        return carry
    lax.fori_loop(0, RPW, eb, 0)

    def db(j, carry):
        pltpu.make_async_copy(ones_v, acc_sh.at[idx_v.at[0]], dsem).wait()
        return carry
    lax.fori_loop(0, RPW, db, 0)
    plsc.subcore_barrier()
    pltpu.sync_copy(acc_sh.at[pl.ds(s * RPS, RPS)],
                    out_hbm.at[c].at[pl.ds(s * RPS, RPS)])


@functools.cache
def _build_agg_kernel():
    return functools.partial(
        pl.kernel,
        out_type=jax.ShapeDtypeStruct((NC, N, HALF), jnp.float32),
        mesh=_sc_mesh(),
        scratch_types=[
            [pltpu.VMEM((RPC // 10, CH2), jnp.int32) for _ in range(2)],
            [pltpu.VMEM((RPC // 10, CH2), jnp.int32) for _ in range(2)],
            [pltpu.VMEM((CH2, HALF), jnp.float32) for _ in range(NB)],
            pltpu.VMEM_SHARED((ACC, HALF), jnp.float32),
            [pltpu.SemaphoreType.DMA for _ in range(NB)],
            [pltpu.SemaphoreType.DMA for _ in range(NB)],
            [pltpu.SemaphoreType.DMA for _ in range(2)],
        ],
    )(_agg_body)


def _agg_body(tab_hbm, src_hbm, dst_hbm, out_hbm, srcb, dstb, bufs,
              acc_sh, gsems, ssems, isems):
    c = lax.axis_index("c")
    s = lax.axis_index("s")

    NSTG = 10
    SRW = RPC // NSTG

    def idx_fetch(h):
        b = h % 2
        off = s * RPC + h * SRW
        pltpu.async_copy(src_hbm.at[pl.ds(off, SRW)], srcb[b], isems[b])
        pltpu.async_copy(dst_hbm.at[pl.ds(off, SRW)], dstb[b], isems[b])

    def idx_wait(h):
        b = h % 2
        off = s * RPC + h * SRW
        pltpu.make_async_copy(src_hbm.at[pl.ds(off, SRW)], srcb[b],
                              isems[b]).wait()
        pltpu.make_async_copy(dst_hbm.at[pl.ds(off, SRW)], dstb[b],
                              isems[b]).wait()

    def g_start(b, r, k):
        pltpu.async_copy(tab_hbm.at[c].at[srcb[b].at[r]], bufs[k], gsems[k])

    def g_wait(k):
        pltpu.make_async_copy(tab_hbm.at[c].at[srcb[0].at[0]], bufs[k],
                              gsems[k]).wait()

    def s_start(b, r, k):
        pltpu.async_copy(bufs[k], acc_sh.at[dstb[b].at[r]], ssems[k],
                         add=True)

    def s_wait(k):
        pltpu.make_async_copy(bufs[k], acc_sh.at[dstb[0].at[0]],
                              ssems[k]).wait()

    idx_fetch(0)
    _zero_2d(bufs[NB - 1], CH2, HALF)
    for i in range(RPS // CH2):
        pltpu.async_copy(bufs[NB - 1],
                         acc_sh.at[pl.ds(s * RPS + i * CH2, CH2)],
                         ssems[NB - 1])
    idx_wait(0)
    for k in range(NB - 1):
        g_start(0, k, k)
    for i in range(RPS // CH2):
        pltpu.make_async_copy(bufs[NB - 1],
                              acc_sh.at[pl.ds(s * RPS, CH2)],
                              ssems[NB - 1]).wait()
    plsc.subcore_barrier()
    g_start(0, NB - 1, NB - 1)
    for h in range(NSTG):
        b = h % 2
        if h + 1 < NSTG:
            idx_fetch(h + 1)

        def rb(i, carry):
            base = i * NB
            for k in range(NB):
                g_wait(k)
                s_start(b, base + k, k)
            for k in range(NB):
                nxt = base + NB + k
                @pl.when(nxt < SRW)
                def _():
                    s_wait(k)
                    g_start(b, nxt, k)
            return carry
        lax.fori_loop(0, SRW // NB, rb, 0)
        if h + 1 < NSTG:
            idx_wait(h + 1)
            for k in range(NB):
                s_wait(k)
                g_start((h + 1) % 2, k, k)
        else:
            for k in range(NB):
                s_wait(k)
    plsc.subcore_barrier()

    @pl.when(s < NS - 1)
    def _copy_full():
        pltpu.sync_copy(acc_sh.at[pl.ds(s * RPS, RPS)],
                        out_hbm.at[c].at[pl.ds(s * RPS, RPS)])

    @pl.when(s == NS - 1)
    def _copy_tail():
        pltpu.sync_copy(acc_sh.at[pl.ds((NS - 1) * RPS, N - (NS - 1) * RPS)],
                        out_hbm.at[c].at[pl.ds((NS - 1) * RPS,
                                               N - (NS - 1) * RPS)])


@functools.cache
def _build_agg1_kernel():
    return functools.partial(
        pl.kernel,
        out_type=jax.ShapeDtypeStruct((NC, ACC), jnp.float32),
        mesh=_sc_mesh(),
        scratch_types=[
            pltpu.VMEM((RPW, CH), jnp.int32),
            pltpu.VMEM((RPW, CH), jnp.int32),
            [pltpu.VMEM((CH,), jnp.float32) for _ in range(NB)],
            pltpu.VMEM((RPS,), jnp.float32),
            pltpu.VMEM_SHARED((ACC,), jnp.float32),
            [pltpu.SemaphoreType.DMA for _ in range(NB)],
            [pltpu.SemaphoreType.DMA for _ in range(NB)],
        ],
    )(_agg1_body)


def _agg1_body(g2_hbm, src_hbm, dst_hbm, out_hbm, src_v, dst_v,
               vals, z_v, acc_sh, gsems, ssems):
    c = lax.axis_index("c")
    s = lax.axis_index("s")
    wid = c * NS + s

    _zero_1d(z_v, RPS)
    pltpu.async_copy(z_v, acc_sh.at[pl.ds(s * RPS, RPS)], ssems[0])
    pltpu.sync_copy(src_hbm.at[pl.ds(wid * RPW, RPW)], src_v)
    pltpu.sync_copy(dst_hbm.at[pl.ds(wid * RPW, RPW)], dst_v)
    pltpu.make_async_copy(z_v, acc_sh.at[pl.ds(s * RPS, RPS)],
                          ssems[0]).wait()
    plsc.subcore_barrier()

    def g_start(r, k):
        pltpu.async_copy(g2_hbm.at[src_v.at[r]], vals[k], gsems[k])

    def g_wait(k):
        pltpu.make_async_copy(g2_hbm.at[src_v.at[0]], vals[k],
                              gsems[k]).wait()

    def s_start(r, k):
        pltpu.async_copy(vals[k], acc_sh.at[dst_v.at[r]], ssems[k],
                         add=True)

    def s_wait(k):
        pltpu.make_async_copy(vals[k], acc_sh.at[dst_v.at[0]],
                              ssems[k]).wait()

    for k in range(NB):
        g_start(k, k)

    def eb(i, carry):
        base = i * NB
        for k in range(NB):
            g_wait(k)
            s_start(base + k, k)
        for k in range(NB):
            nxt = base + NB + k
            @pl.when(nxt < RPW)
            def _():
                s_wait(k)
                g_start(nxt, k)
        return carry
    lax.fori_loop(0, RPW // NB, eb, 0)
    for k in range(NB):
        s_wait(k)
    plsc.subcore_barrier()
    pltpu.sync_copy(acc_sh.at[pl.ds(s * RPS, RPS)],
                    out_hbm.at[c].at[pl.ds(s * RPS, RPS)])


def _mm_body(x_ref, w_ref, d0_ref, d1_ref, tab_ref, dinv_ref):
    deg = d0_ref[...] + d1_ref[...] + 1.0
    dinv = lax.rsqrt(deg)
    h = jnp.dot(x_ref[...], w_ref[...], preferred_element_type=jnp.float32)
    g = h * dinv
    tab_ref[0] = g[:, :HALF]
    tab_ref[1] = g[:, HALF:]
    dinv_ref[...] = dinv


def _ep_body(s_ref, g_ref, dinv_ref, b1_ref, w2_ref, g2_ref):
    sfull = jnp.concatenate([s_ref[0], s_ref[1]], axis=1)
    gfull = jnp.concatenate([g_ref[0], g_ref[1]], axis=1)
    dinv = dinv_ref[...]
    o1 = jnp.maximum(dinv * (sfull + gfull) + b1_ref[...], 0.0)
    h2 = jnp.dot(o1, w2_ref[...], preferred_element_type=jnp.float32)
    g2_ref[...] = h2 * dinv


def _fin_body(g2_ref, sa_ref, sb_ref, dinv_ref, b2_ref, out_ref):
    t = dinv_ref[...] * (sa_ref[...] + sb_ref[...] + g2_ref[...]) + b2_ref[...]
    out_ref[...] = jax.nn.sigmoid(t)


def kernel(x, edge_index, W1, b1, W2, b2):
    src = edge_index[0].astype(jnp.int32)
    dst = edge_index[1].astype(jnp.int32)
    pad = EP - E
    srcp = jnp.concatenate([src, jnp.zeros((pad,), jnp.int32)]).reshape(NROW, CH)
    dstp = jnp.concatenate([dst, jnp.full((pad,), DUMMY, jnp.int32)]).reshape(NROW, CH)

    deg2 = _build_deg_kernel()(dstp)
    d0 = deg2[0].reshape(ACC, 1)
    d1 = deg2[1].reshape(ACC, 1)

    tab, dinv = pl.pallas_call(
        _mm_body,
        grid=(N // R,),
        in_specs=[
            pl.BlockSpec((R, D), lambda i: (i, 0)),
            pl.BlockSpec((D, D), lambda i: (0, 0)),
            pl.BlockSpec((R, 1), lambda i: (i, 0)),
            pl.BlockSpec((R, 1), lambda i: (i, 0)),
        ],
        out_specs=[
            pl.BlockSpec((2, R, HALF), lambda i: (0, i, 0)),
            pl.BlockSpec((R, 1), lambda i: (i, 0)),
        ],
        out_shape=[
            jax.ShapeDtypeStruct((2, N, HALF), jnp.float32),
            jax.ShapeDtypeStruct((N, 1), jnp.float32),
        ],
    )(x, W1, d0, d1)

    sacc = _build_agg_kernel()(tab, srcp.reshape(NROW2, CH2),
                               dstp.reshape(NROW2, CH2))

    g2 = pl.pallas_call(
        _ep_body,
        grid=(N // R,),
        in_specs=[
            pl.BlockSpec((2, R, HALF), lambda i: (0, i, 0)),
            pl.BlockSpec((2, R, HALF), lambda i: (0, i, 0)),
            pl.BlockSpec((R, 1), lambda i: (i, 0)),
            pl.BlockSpec((1, D), lambda i: (0, 0)),
            pl.BlockSpec((D, 1), lambda i: (0, 0)),
        ],
        out_specs=pl.BlockSpec((R, 1), lambda i: (i, 0)),
        out_shape=jax.ShapeDtypeStruct((N, 1), jnp.float32),
    )(sacc, tab, dinv, b1.reshape(1, D), W2)

    s2 = _build_agg1_kernel()(g2.reshape(N), srcp, dstp)

    out = pl.pallas_call(
        _fin_body,
        grid=(N // R,),
        in_specs=[
            pl.BlockSpec((R, 1), lambda i: (i, 0)),
            pl.BlockSpec((R, 1), lambda i: (i, 0)),
            pl.BlockSpec((R, 1), lambda i: (i, 0)),
            pl.BlockSpec((R, 1), lambda i: (i, 0)),
            pl.BlockSpec((1, 1), lambda i: (0, 0)),
        ],
        out_specs=pl.BlockSpec((R, 1), lambda i: (i, 0)),
        out_shape=jax.ShapeDtypeStruct((N, 1), jnp.float32),
    )(g2, s2[0].reshape(ACC, 1), s2[1].reshape(ACC, 1), dinv,
      b2.reshape(1, 1))
    return out
